# baseline (speedup 1.0000x reference)
import functools

import jax
import jax.numpy as jnp
from jax import lax
from jax.experimental import pallas as pl
from jax.experimental.pallas import tpu as pltpu
from jax.experimental.pallas import tpu_sc as plsc

_NC = 2
_NS = 16
_NW = _NC * _NS
_CH = 128


def kernel(input_ids, table):
    B, S = input_ids.shape
    V, D = table.shape
    N = B * S
    assert N % (_NW * _CH) == 0
    per_w = N // _NW
    n_chunks = per_w // _CH

    ids = input_ids.reshape(_NW, n_chunks, _CH).astype(jnp.int32)
    mesh = plsc.VectorSubcoreMesh(
        core_axis_name="c", subcore_axis_name="s", num_cores=_NC, num_subcores=_NS
    )

    @functools.partial(
        pl.kernel,
        out_type=jax.ShapeDtypeStruct((N, D), jnp.float32),
        mesh=mesh,
        scratch_types=[
            pltpu.VMEM((n_chunks, _CH), jnp.int32),
            pltpu.VMEM((_CH, D), jnp.float32),
            pltpu.SemaphoreType.DMA,
        ],
        compiler_params=pltpu.CompilerParams(use_tc_tiling_on_sc=False),
    )
    def emb(ids_hbm, table_hbm, out_hbm, idx_v, rows_v, sem):
        wid = lax.axis_index("s") * _NC + lax.axis_index("c")
        pltpu.sync_copy(ids_hbm.at[wid], idx_v)

        def chunk(j, carry):
            pltpu.async_copy(table_hbm.at[idx_v.at[j]], rows_v, sem).wait()
            pltpu.sync_copy(rows_v, out_hbm.at[pl.ds((wid * n_chunks + j) * _CH, _CH)])
            return carry

        lax.fori_loop(0, n_chunks, chunk, 0)

    out = emb(ids, table)
    return out.reshape(B, S, D)


# speedup vs baseline: 1.1187x; 1.1187x over previous
import functools

import jax
import jax.numpy as jnp
from jax import lax
from jax.experimental import pallas as pl
from jax.experimental.pallas import tpu as pltpu
from jax.experimental.pallas import tpu_sc as plsc

_NC = 2
_NS = 16
_NW = _NC * _NS
_CH = 128
_NBUF = 4


def kernel(input_ids, table):
    B, S = input_ids.shape
    V, D = table.shape
    N = B * S
    assert N % (_NW * _CH) == 0
    per_w = N // _NW
    n_chunks = per_w // _CH
    assert n_chunks % _NBUF == 0 and n_chunks >= 2 * _NBUF

    ids = input_ids.reshape(_NW, n_chunks, _CH).astype(jnp.int32)
    mesh = plsc.VectorSubcoreMesh(
        core_axis_name="c", subcore_axis_name="s", num_cores=_NC, num_subcores=_NS
    )

    @functools.partial(
        pl.kernel,
        out_type=jax.ShapeDtypeStruct((N, D), jnp.float32),
        mesh=mesh,
        scratch_types=[
            pltpu.VMEM((n_chunks, _CH), jnp.int32),
            pltpu.VMEM((_NBUF, _CH, D), jnp.float32),
            pltpu.SemaphoreType.DMA((_NBUF,)),
            pltpu.SemaphoreType.DMA((_NBUF,)),
        ],
        compiler_params=pltpu.CompilerParams(use_tc_tiling_on_sc=False),
    )
    def emb(ids_hbm, table_hbm, out_hbm, idx_v, rows_v, sem_g, sem_w):
        wid = lax.axis_index("s") * _NC + lax.axis_index("c")
        base = wid * n_chunks
        pltpu.sync_copy(ids_hbm.at[wid], idx_v)

        def fire_gather(j, b):
            pltpu.async_copy(table_hbm.at[idx_v.at[j]], rows_v.at[b], sem_g.at[b])

        def fire_write(j, b):
            pltpu.async_copy(
                rows_v.at[b], out_hbm.at[pl.ds((base + j) * _CH, _CH)], sem_w.at[b]
            )

        def wait_gather(j, b):
            pltpu.make_async_copy(
                table_hbm.at[idx_v.at[j]], rows_v.at[b], sem_g.at[b]
            ).wait()

        def wait_write(j, b):
            pltpu.make_async_copy(
                rows_v.at[b], out_hbm.at[pl.ds((base + j) * _CH, _CH)], sem_w.at[b]
            ).wait()

        fire_gather(0, 0)
        fire_gather(1, 1)

        def step(j2, carry):
            for b in range(_NBUF):
                j = j2 * _NBUF + b
                bg = (b + 2) % _NBUF

                @pl.when(j >= 2)
                def _():
                    wait_write(j - 2, bg)

                @pl.when(j + 2 < n_chunks)
                def _():
                    fire_gather(j + 2, bg)

                wait_gather(j, b)
                fire_write(j, b)
            return carry

        lax.fori_loop(0, n_chunks // _NBUF, step, 0)
        wait_write(n_chunks - 2, (n_chunks - 2) % _NBUF)
        wait_write(n_chunks - 1, (n_chunks - 1) % _NBUF)

    out = emb(ids, table)
    return out.reshape(B, S, D)
